# baseline (speedup 1.0000x reference)
import functools

import jax
import jax.numpy as jnp
from jax import lax
from jax.experimental import pallas as pl
from jax.experimental.pallas import tpu as pltpu
from jax.experimental.pallas import tpu_sc as plsc

B, T, Q = 8, 2048, 128
P = 1000000
MAX_PATH = 20

NC, NS = 2, 16
NW = NC * NS
ROWS_PER_W = T // NW
HALF_ROWS = 32

UNROLL = 4
CHUNK = 8192
P_PAD = 1015808
NCHUNK = P_PAD // CHUNK
VEC_PER_CHUNK = CHUNK // 16


def _pack_body(n1_ref, n2_ref, d_ref, o_ref):
    o_ref[...] = (
        (n1_ref[...] << 16) | (n2_ref[...] << 5) | d_ref[...]
    )


def _bcast_body(p_ref, o_ref):
    o_ref[...] = jnp.broadcast_to(p_ref[...][None], o_ref.shape)


def _sc_body(packed_hbm, emb_hbm, plane_hbm, buf, scanbuf, embv, sems, osem):
    wid = lax.axis_index("s") * NC + lax.axis_index("c")
    row0 = wid * ROWS_PER_W

    pltpu.sync_copy(emb_hbm, embv)
    cvec = plsc.load_gather(embv, [jnp.full((16,), MAX_PATH - 1, jnp.int32)])

    def _start(c):
        slot = c & 1
        return pltpu.async_copy(
            packed_hbm.at[pl.ds(c * CHUNK, CHUNK)], scanbuf.at[slot],
            sems.at[slot])

    for h in range(2):
        rowbase = row0 + h * HALF_ROWS

        def _init_row(i, _):
            def _init_v(j, _):
                for u in range(8):
                    buf[i, pl.ds(j * 128 + u * 16, 16)] = cvec
                return 0
            lax.fori_loop(0, T // 128, _init_v, 0)
            return 0

        lax.fori_loop(0, HALF_ROWS, _init_row, 0)

        _start(0)

        def _chunk(c, _):
            slot = c & 1

            @pl.when(c + 1 < NCHUNK)
            def _():
                _start(c + 1)

            pltpu.make_async_copy(
                packed_hbm.at[pl.ds(c * CHUNK, CHUNK)], scanbuf.at[slot],
                sems.at[slot]).wait()

            def _vec(j, _):
                for u in range(UNROLL):
                    v = scanbuf[slot, pl.ds((j * UNROLL + u) * 16, 16)]
                    rh = lax.shift_right_logical(v, 16) - rowbase
                    m = (rh >= 0) & (rh < HALF_ROWS)
                    n2 = lax.shift_right_logical(v, 5) & 0x7FF
                    d = v & 0x1F
                    val = plsc.load_gather(embv, [d])
                    plsc.store_scatter(buf, [rh, n2], val, mask=m)
                return 0

            lax.fori_loop(0, VEC_PER_CHUNK // UNROLL, _vec, 0)
            return 0

        lax.fori_loop(0, NCHUNK, _chunk, 0)

        pltpu.async_copy(
            buf, plane_hbm.at[pl.ds(rowbase, HALF_ROWS), :], osem)
        pltpu.make_async_copy(
            buf, plane_hbm.at[pl.ds(rowbase, HALF_ROWS), :], osem).wait()


@jax.jit
def kernel(x, path_distance_map, edge_index_map, distance_embedding):
    del x
    n1 = edge_index_map[:, 0]
    n2 = edge_index_map[:, 1]
    d = path_distance_map
    pad = P_PAD - P
    n1 = jnp.concatenate([n1, jnp.full((pad,), T, jnp.int32)]).reshape(-1, 128)
    n2 = jnp.concatenate([n2, jnp.zeros((pad,), jnp.int32)]).reshape(-1, 128)
    d = jnp.concatenate([d, jnp.zeros((pad,), jnp.int32)]).reshape(-1, 128)

    rows = P_PAD // 128
    blk = rows // 8
    packed = pl.pallas_call(
        _pack_body,
        out_shape=jax.ShapeDtypeStruct((rows, 128), jnp.int32),
        grid=(8,),
        in_specs=[pl.BlockSpec((blk, 128), lambda i: (i, 0))] * 3,
        out_specs=pl.BlockSpec((blk, 128), lambda i: (i, 0)),
    )(n1, n2, d).reshape(P_PAD)

    emb32 = jnp.concatenate(
        [distance_embedding[:, 0],
         jnp.zeros((32 - MAX_PATH,), jnp.float32)])

    sc = pl.kernel(
        _sc_body,
        out_type=jax.ShapeDtypeStruct((T, T), jnp.float32),
        mesh=plsc.VectorSubcoreMesh(core_axis_name="c", subcore_axis_name="s"),
        scratch_types=[
            pltpu.VMEM((HALF_ROWS, T), jnp.float32),
            pltpu.VMEM((2, CHUNK), jnp.int32),
            pltpu.VMEM((32,), jnp.float32),
            pltpu.SemaphoreType.DMA((2,)),
            pltpu.SemaphoreType.DMA,
        ],
        compiler_params=pltpu.CompilerParams(needs_layout_passes=False),
    )
    plane = sc(packed, emb32)

    RB = 256
    return pl.pallas_call(
        _bcast_body,
        out_shape=jax.ShapeDtypeStruct((B, T, T), jnp.float32),
        grid=(T // RB,),
        in_specs=[pl.BlockSpec((RB, T), lambda i: (i, 0))],
        out_specs=pl.BlockSpec((B, RB, T), lambda i: (0, i, 0)),
    )(plane)


# speedup vs baseline: 1.2097x; 1.2097x over previous
import functools

import jax
import jax.numpy as jnp
from jax import lax
from jax.experimental import pallas as pl
from jax.experimental.pallas import tpu as pltpu
from jax.experimental.pallas import tpu_sc as plsc

B, T, Q = 8, 2048, 128
P = 1000000
MAX_PATH = 20

NC, NS = 2, 16
NW = NC * NS
ROWS_PER_W = T // NW
HALF_ROWS = 32

CHUNK = 4096
P_PAD = 1015808
NCHUNK = P_PAD // CHUNK
VEC_PER_CHUNK = CHUNK // 16
GROUPS = VEC_PER_CHUNK // 16

CAP = 48000


def _pack_body(n1_ref, n2_ref, d_ref, o_ref):
    o_ref[...] = (
        (n1_ref[...] << 16) | (n2_ref[...] << 5) | d_ref[...]
    )


def _bcast_body(p_ref, o_ref):
    o_ref[...] = jnp.broadcast_to(p_ref[...][None], o_ref.shape)


def _sc_body(packed_hbm, emb_hbm, plane_hbm, plist, buf, scanbuf, staging,
             countbuf, embv, sems, osem):
    wid = lax.axis_index("s") * NC + lax.axis_index("c")
    row0 = wid * ROWS_PER_W

    pltpu.sync_copy(emb_hbm, embv)
    cvec = plsc.load_gather(embv, [jnp.full((16,), MAX_PATH - 1, jnp.int32)])
    iota = lax.iota(jnp.int32, 16)
    zero16 = jnp.zeros((16,), jnp.int32)

    neg1 = jnp.full((16,), -1, jnp.int32)

    def _pf(i, _):
        plist[pl.ds(i * 16, 16)] = neg1
        return 0

    lax.fori_loop(0, CAP // 16, _pf, 0)

    def _start(c):
        slot = c & 1
        return pltpu.async_copy(
            packed_hbm.at[pl.ds(c * CHUNK, CHUNK)], scanbuf.at[slot],
            sems.at[slot])

    _start(0)

    def _chunk(c, cntv):
        slot = c & 1

        @pl.when(c + 1 < NCHUNK)
        def _():
            _start(c + 1)

        pltpu.make_async_copy(
            packed_hbm.at[pl.ds(c * CHUNK, CHUNK)], scanbuf.at[slot],
            sems.at[slot]).wait()

        def _grp(g, _):
            cnts = zero16
            for u in range(16):
                v = scanbuf[slot, pl.ds((g * 16 + u) * 16, 16)]
                r = lax.shift_right_logical(v, 16) - row0
                m = (r >= 0) & (r < ROWS_PER_W)
                plsc.store_compressed(
                    staging.at[pl.ds((g * 16 + u) * 16, 16)], v, mask=m)
                pc = plsc.all_reduce_population_count(m)
                cnts = jnp.where(iota == u, pc, cnts)
            countbuf[pl.ds(g * 16, 16)] = cnts
            return 0

        lax.fori_loop(0, GROUPS, _grp, 0)

        def _mov(g, cntv):
            for u in range(16):
                s = g * 16 + u
                pc = plsc.load_gather(countbuf, [zero16 + s])
                sv = staging[pl.ds(s * 16, 16)]
                m2 = iota < pc
                plsc.store_scatter(plist, [cntv + iota], sv, mask=m2)
                cntv = jnp.minimum(cntv + pc, CAP - 16)
            return cntv

        return lax.fori_loop(0, GROUPS, _mov, cntv)

    cntv = lax.fori_loop(0, NCHUNK, _chunk, zero16)
    nvec = (cntv[0] + 15) // 16

    for h in range(2):
        rowbase = row0 + h * HALF_ROWS

        def _init_row(i, _):
            def _init_v(j, _):
                for u in range(8):
                    buf[i, pl.ds(j * 128 + u * 16, 16)] = cvec
                return 0
            lax.fori_loop(0, T // 128, _init_v, 0)
            return 0

        lax.fori_loop(0, HALF_ROWS, _init_row, 0)

        def _scat(j, _):
            v = plist[pl.ds(j * 16, 16)]
            rh = lax.shift_right_logical(v, 16) - rowbase
            m = (rh >= 0) & (rh < HALF_ROWS)
            n2 = lax.shift_right_logical(v, 5) & 0x7FF
            d = v & 0x1F
            val = plsc.load_gather(embv, [d])
            plsc.store_scatter(buf, [rh, n2], val, mask=m)
            return 0

        lax.fori_loop(0, nvec, _scat, 0)

        pltpu.async_copy(
            buf, plane_hbm.at[pl.ds(rowbase, HALF_ROWS), :], osem)
        pltpu.make_async_copy(
            buf, plane_hbm.at[pl.ds(rowbase, HALF_ROWS), :], osem).wait()


@jax.jit
def kernel(x, path_distance_map, edge_index_map, distance_embedding):
    del x
    n1 = edge_index_map[:, 0]
    n2 = edge_index_map[:, 1]
    d = path_distance_map
    pad = P_PAD - P
    n1 = jnp.concatenate([n1, jnp.full((pad,), T, jnp.int32)]).reshape(-1, 128)
    n2 = jnp.concatenate([n2, jnp.zeros((pad,), jnp.int32)]).reshape(-1, 128)
    d = jnp.concatenate([d, jnp.zeros((pad,), jnp.int32)]).reshape(-1, 128)

    rows = P_PAD // 128
    blk = rows // 8
    packed = pl.pallas_call(
        _pack_body,
        out_shape=jax.ShapeDtypeStruct((rows, 128), jnp.int32),
        grid=(8,),
        in_specs=[pl.BlockSpec((blk, 128), lambda i: (i, 0))] * 3,
        out_specs=pl.BlockSpec((blk, 128), lambda i: (i, 0)),
    )(n1, n2, d).reshape(P_PAD)

    emb32 = jnp.concatenate(
        [distance_embedding[:, 0],
         jnp.zeros((32 - MAX_PATH,), jnp.float32)])

    sc = pl.kernel(
        _sc_body,
        out_type=jax.ShapeDtypeStruct((T, T), jnp.float32),
        mesh=plsc.VectorSubcoreMesh(core_axis_name="c", subcore_axis_name="s"),
        scratch_types=[
            pltpu.VMEM((CAP,), jnp.int32),
            pltpu.VMEM((HALF_ROWS, T), jnp.float32),
            pltpu.VMEM((2, CHUNK), jnp.int32),
            pltpu.VMEM((CHUNK,), jnp.int32),
            pltpu.VMEM((VEC_PER_CHUNK,), jnp.int32),
            pltpu.VMEM((32,), jnp.float32),
            pltpu.SemaphoreType.DMA((2,)),
            pltpu.SemaphoreType.DMA,
        ],
        compiler_params=pltpu.CompilerParams(needs_layout_passes=False),
    )
    plane = sc(packed, emb32)

    RB = 256
    return pl.pallas_call(
        _bcast_body,
        out_shape=jax.ShapeDtypeStruct((B, T, T), jnp.float32),
        grid=(T // RB,),
        in_specs=[pl.BlockSpec((RB, T), lambda i: (i, 0))],
        out_specs=pl.BlockSpec((B, RB, T), lambda i: (0, i, 0)),
    )(plane)


# speedup vs baseline: 2.6824x; 2.2174x over previous
import functools

import jax
import jax.numpy as jnp
from jax import lax
from jax.experimental import pallas as pl
from jax.experimental.pallas import tpu as pltpu
from jax.experimental.pallas import tpu_sc as plsc

B, T, Q = 8, 2048, 128
P = 1000000
MAX_PATH = 20

NC, NS = 2, 16
NW = NC * NS
ROWS_PER_W = T // NW
HALF_ROWS = 32

CHUNK = 4096
P_PAD = 1015808
NCHUNK = P_PAD // CHUNK
VEC_PER_CHUNK = CHUNK // 16
GROUPS = VEC_PER_CHUNK // 16

CAP = 48128


def _pack_body(n1_ref, n2_ref, d_ref, o_ref):
    o_ref[...] = (
        (n1_ref[...] << 16) | (n2_ref[...] << 5) | d_ref[...]
    )


def _bcast_body(p_ref, o_ref):
    o_ref[...] = jnp.broadcast_to(p_ref[...][None], o_ref.shape)


def _splat(x, u):
    return lax.gather(
        x, jnp.full((16, 1), u, jnp.int32),
        lax.GatherDimensionNumbers(
            offset_dims=(), collapsed_slice_dims=(0,), start_index_map=(0,)),
        (1,), mode=lax.GatherScatterMode.PROMISE_IN_BOUNDS)


def _sc_body(packed_hbm, emb_hbm, plane_hbm, plist, buf, scanbuf, staging,
             countbuf, embv, sems, osem):
    wid = lax.axis_index("s") * NC + lax.axis_index("c")
    row0 = wid * ROWS_PER_W

    pltpu.sync_copy(emb_hbm, embv)
    cvec = plsc.load_gather(embv, [jnp.full((16,), MAX_PATH - 1, jnp.int32)])
    iota = lax.iota(jnp.int32, 16)
    zero16 = jnp.zeros((16,), jnp.int32)

    neg1 = jnp.full((16,), -1, jnp.int32)

    def _pf(i, _):
        plist[pl.ds(i * 16, 16)] = neg1
        return 0

    lax.fori_loop(0, CAP // 16, _pf, 0)

    def _start(c):
        slot = c & 1
        return pltpu.async_copy(
            packed_hbm.at[pl.ds(c * CHUNK, CHUNK)], scanbuf.at[slot],
            sems.at[slot])

    _start(0)

    def _chunk(c, cntv):
        slot = c & 1

        @pl.when(c + 1 < NCHUNK)
        def _():
            _start(c + 1)

        pltpu.make_async_copy(
            packed_hbm.at[pl.ds(c * CHUNK, CHUNK)], scanbuf.at[slot],
            sems.at[slot]).wait()

        lo = row0 * 65536

        def _grp(g, _):
            vs = [scanbuf[slot, pl.ds((g * 16 + u) * 16, 16)]
                  for u in range(16)]
            ms = [(vs[u] - lo).astype(jnp.uint32)
                  < jnp.uint32(ROWS_PER_W * 65536) for u in range(16)]
            cnts = zero16
            for u in range(16):
                plsc.store_compressed(
                    staging.at[pl.ds((g * 16 + u) * 16, 16)], vs[u],
                    mask=ms[u])
                pc = plsc.all_reduce_population_count(ms[u])
                cnts = jnp.where(iota == u, pc, cnts)
            countbuf[pl.ds(g * 16, 16)] = cnts
            return 0

        lax.fori_loop(0, GROUPS, _grp, 0)

        def _mov(g, cntv):
            cnts = countbuf[pl.ds(g * 16, 16)]
            incl = plsc.cumsum(cnts)
            excl = incl - cnts
            svs = [staging[pl.ds((g * 16 + u) * 16, 16)] for u in range(16)]
            for u in range(16):
                base = jnp.minimum(cntv + _splat(excl, u), CAP - 16)
                m2 = iota < _splat(cnts, u)
                plsc.store_scatter(plist, [base + iota], svs[u], mask=m2)
            return jnp.minimum(cntv + _splat(incl, 15), CAP - 16)

        return lax.fori_loop(0, GROUPS, _mov, cntv)

    cntv = lax.fori_loop(0, NCHUNK, _chunk, zero16)
    ngrp = (cntv[0] + 255) // 256

    for h in range(2):
        rowbase = row0 + h * HALF_ROWS

        def _init_row(i, _):
            def _init_v(j, _):
                for u in range(8):
                    buf[i, pl.ds(j * 128 + u * 16, 16)] = cvec
                return 0
            lax.fori_loop(0, T // 128, _init_v, 0)
            return 0

        lax.fori_loop(0, HALF_ROWS, _init_row, 0)

        def _scat(j, _):
            vs = [plist[pl.ds((j * 16 + u) * 16, 16)] for u in range(16)]
            rhs = [lax.shift_right_logical(vs[u], 16) - rowbase
                   for u in range(16)]
            ms = [rhs[u].astype(jnp.uint32) < jnp.uint32(HALF_ROWS)
                  for u in range(16)]
            n2s = [lax.shift_right_logical(vs[u], 5) & 0x7FF
                   for u in range(16)]
            vals = [plsc.load_gather(embv, [vs[u] & 0x1F])
                    for u in range(16)]
            for u in range(16):
                plsc.store_scatter(buf, [rhs[u], n2s[u]], vals[u],
                                   mask=ms[u])
            return 0

        lax.fori_loop(0, ngrp, _scat, 0)

        pltpu.async_copy(
            buf, plane_hbm.at[pl.ds(rowbase, HALF_ROWS), :], osem)
        pltpu.make_async_copy(
            buf, plane_hbm.at[pl.ds(rowbase, HALF_ROWS), :], osem).wait()


@jax.jit
def kernel(x, path_distance_map, edge_index_map, distance_embedding):
    del x
    n1 = edge_index_map[:, 0]
    n2 = edge_index_map[:, 1]
    d = path_distance_map
    pad = P_PAD - P
    n1 = jnp.concatenate([n1, jnp.full((pad,), T, jnp.int32)]).reshape(-1, 128)
    n2 = jnp.concatenate([n2, jnp.zeros((pad,), jnp.int32)]).reshape(-1, 128)
    d = jnp.concatenate([d, jnp.zeros((pad,), jnp.int32)]).reshape(-1, 128)

    rows = P_PAD // 128
    blk = rows // 8
    packed = pl.pallas_call(
        _pack_body,
        out_shape=jax.ShapeDtypeStruct((rows, 128), jnp.int32),
        grid=(8,),
        in_specs=[pl.BlockSpec((blk, 128), lambda i: (i, 0))] * 3,
        out_specs=pl.BlockSpec((blk, 128), lambda i: (i, 0)),
    )(n1, n2, d).reshape(P_PAD)

    emb32 = jnp.concatenate(
        [distance_embedding[:, 0],
         jnp.zeros((32 - MAX_PATH,), jnp.float32)])

    sc = pl.kernel(
        _sc_body,
        out_type=jax.ShapeDtypeStruct((T, T), jnp.float32),
        mesh=plsc.VectorSubcoreMesh(core_axis_name="c", subcore_axis_name="s"),
        scratch_types=[
            pltpu.VMEM((CAP,), jnp.int32),
            pltpu.VMEM((HALF_ROWS, T), jnp.float32),
            pltpu.VMEM((2, CHUNK), jnp.int32),
            pltpu.VMEM((CHUNK,), jnp.int32),
            pltpu.VMEM((VEC_PER_CHUNK,), jnp.int32),
            pltpu.VMEM((32,), jnp.float32),
            pltpu.SemaphoreType.DMA((2,)),
            pltpu.SemaphoreType.DMA,
        ],
        compiler_params=pltpu.CompilerParams(needs_layout_passes=False),
    )
    plane = sc(packed, emb32)

    RB = 256
    return pl.pallas_call(
        _bcast_body,
        out_shape=jax.ShapeDtypeStruct((B, T, T), jnp.float32),
        grid=(T // RB,),
        in_specs=[pl.BlockSpec((RB, T), lambda i: (i, 0))],
        out_specs=pl.BlockSpec((B, RB, T), lambda i: (0, i, 0)),
    )(plane)


# speedup vs baseline: 3.3903x; 1.2639x over previous
import functools

import jax
import jax.numpy as jnp
from jax import lax
from jax.experimental import pallas as pl
from jax.experimental.pallas import tpu as pltpu
from jax.experimental.pallas import tpu_sc as plsc

B, T, Q = 8, 2048, 128
P = 1000000
MAX_PATH = 20

NC, NS = 2, 16
NW = NC * NS
ROWS_PER_W = T // NW
HALF_ROWS = 32

CHUNK = 4096
P_PAD = 1015808
NCHUNK = P_PAD // CHUNK
VEC_PER_CHUNK = CHUNK // 16
GROUPS = VEC_PER_CHUNK // 16

CAP = 48128


def _pack_body(n1_ref, n2_ref, d_ref, o_ref):
    o_ref[...] = (
        (n1_ref[...] << 16) | (n2_ref[...] << 5) | d_ref[...]
    )


def _bcast_body(p_ref, o_ref):
    o_ref[...] = jnp.broadcast_to(p_ref[...][None], o_ref.shape)


def _splat(x, u):
    return lax.gather(
        x, jnp.full((16, 1), u, jnp.int32),
        lax.GatherDimensionNumbers(
            offset_dims=(), collapsed_slice_dims=(0,), start_index_map=(0,)),
        (1,), mode=lax.GatherScatterMode.PROMISE_IN_BOUNDS)


def _sc_body(packed_hbm, emb_hbm, plane_hbm, plist, buf, scanbuf, staging,
             countbuf, embv, sems, osem):
    wid = lax.axis_index("s") * NC + lax.axis_index("c")
    row0 = wid * ROWS_PER_W

    pltpu.sync_copy(emb_hbm, embv)
    cvec = plsc.load_gather(embv, [jnp.full((16,), MAX_PATH - 1, jnp.int32)])
    iota = lax.iota(jnp.int32, 16)
    zero16 = jnp.zeros((16,), jnp.int32)

    neg1 = jnp.full((16,), -1, jnp.int32)

    def _pf(i, _):
        plist[pl.ds(i * 16, 16)] = neg1
        return 0

    lax.fori_loop(0, CAP // 16, _pf, 0)

    def _start(c):
        slot = c & 1
        return pltpu.async_copy(
            packed_hbm.at[pl.ds(c * CHUNK, CHUNK)], scanbuf.at[slot],
            sems.at[slot])

    _start(0)

    def _chunk(c, cntv):
        slot = c & 1

        @pl.when(c + 1 < NCHUNK)
        def _():
            _start(c + 1)

        pltpu.make_async_copy(
            packed_hbm.at[pl.ds(c * CHUNK, CHUNK)], scanbuf.at[slot],
            sems.at[slot]).wait()

        lo = row0 * 65536

        def _grp(g, cntv):
            vs = [scanbuf[slot, pl.ds((g * 16 + u) * 16, 16)]
                  for u in range(16)]
            ms = [(vs[u] - lo).astype(jnp.uint32)
                  < jnp.uint32(ROWS_PER_W * 65536) for u in range(16)]
            cnts = zero16
            for u in range(16):
                pc = plsc.all_reduce_population_count(ms[u])
                cnts = jnp.where(iota == u, pc, cnts)
            incl = plsc.cumsum(cnts)
            excl = incl - cnts
            for u in range(16):
                rank = plsc.cumsum(ms[u].astype(jnp.int32)) - 1
                base = jnp.minimum(cntv + _splat(excl, u), CAP - 16)
                plsc.store_scatter(plist, [base + rank], vs[u], mask=ms[u])
            return jnp.minimum(cntv + _splat(incl, 15), CAP - 16)

        return lax.fori_loop(0, GROUPS, _grp, cntv)

    cntv = lax.fori_loop(0, NCHUNK, _chunk, zero16)
    ngrp = (cntv[0] + 255) // 256

    for h in range(2):
        rowbase = row0 + h * HALF_ROWS

        def _init_row(i, _):
            def _init_v(j, _):
                for u in range(8):
                    buf[i, pl.ds(j * 128 + u * 16, 16)] = cvec
                return 0
            lax.fori_loop(0, T // 128, _init_v, 0)
            return 0

        lax.fori_loop(0, HALF_ROWS, _init_row, 0)

        def _scat(j, _):
            vs = [plist[pl.ds((j * 16 + u) * 16, 16)] for u in range(16)]
            rhs = [lax.shift_right_logical(vs[u], 16) - rowbase
                   for u in range(16)]
            ms = [rhs[u].astype(jnp.uint32) < jnp.uint32(HALF_ROWS)
                  for u in range(16)]
            n2s = [lax.shift_right_logical(vs[u], 5) & 0x7FF
                   for u in range(16)]
            vals = [plsc.load_gather(embv, [vs[u] & 0x1F])
                    for u in range(16)]
            for u in range(16):
                plsc.store_scatter(buf, [rhs[u], n2s[u]], vals[u],
                                   mask=ms[u])
            return 0

        lax.fori_loop(0, ngrp, _scat, 0)

        pltpu.async_copy(
            buf, plane_hbm.at[pl.ds(rowbase, HALF_ROWS), :], osem)
        pltpu.make_async_copy(
            buf, plane_hbm.at[pl.ds(rowbase, HALF_ROWS), :], osem).wait()


@jax.jit
def kernel(x, path_distance_map, edge_index_map, distance_embedding):
    del x
    n1 = edge_index_map[:, 0]
    n2 = edge_index_map[:, 1]
    d = path_distance_map
    pad = P_PAD - P
    n1 = jnp.concatenate([n1, jnp.full((pad,), T, jnp.int32)]).reshape(-1, 128)
    n2 = jnp.concatenate([n2, jnp.zeros((pad,), jnp.int32)]).reshape(-1, 128)
    d = jnp.concatenate([d, jnp.zeros((pad,), jnp.int32)]).reshape(-1, 128)

    rows = P_PAD // 128
    blk = rows // 8
    packed = pl.pallas_call(
        _pack_body,
        out_shape=jax.ShapeDtypeStruct((rows, 128), jnp.int32),
        grid=(8,),
        in_specs=[pl.BlockSpec((blk, 128), lambda i: (i, 0))] * 3,
        out_specs=pl.BlockSpec((blk, 128), lambda i: (i, 0)),
    )(n1, n2, d).reshape(P_PAD)

    emb32 = jnp.concatenate(
        [distance_embedding[:, 0],
         jnp.zeros((32 - MAX_PATH,), jnp.float32)])

    sc = pl.kernel(
        _sc_body,
        out_type=jax.ShapeDtypeStruct((T, T), jnp.float32),
        mesh=plsc.VectorSubcoreMesh(core_axis_name="c", subcore_axis_name="s"),
        scratch_types=[
            pltpu.VMEM((CAP,), jnp.int32),
            pltpu.VMEM((HALF_ROWS, T), jnp.float32),
            pltpu.VMEM((2, CHUNK), jnp.int32),
            pltpu.VMEM((CHUNK,), jnp.int32),
            pltpu.VMEM((VEC_PER_CHUNK,), jnp.int32),
            pltpu.VMEM((32,), jnp.float32),
            pltpu.SemaphoreType.DMA((2,)),
            pltpu.SemaphoreType.DMA,
        ],
        compiler_params=pltpu.CompilerParams(needs_layout_passes=False),
    )
    plane = sc(packed, emb32)

    RB = 256
    return pl.pallas_call(
        _bcast_body,
        out_shape=jax.ShapeDtypeStruct((B, T, T), jnp.float32),
        grid=(T // RB,),
        in_specs=[pl.BlockSpec((RB, T), lambda i: (i, 0))],
        out_specs=pl.BlockSpec((B, RB, T), lambda i: (0, i, 0)),
    )(plane)


# speedup vs baseline: 3.3915x; 1.0004x over previous
import functools

import jax
import jax.numpy as jnp
from jax import lax
from jax.experimental import pallas as pl
from jax.experimental.pallas import tpu as pltpu
from jax.experimental.pallas import tpu_sc as plsc

B, T, Q = 8, 2048, 128
P = 1000000
MAX_PATH = 20

NC, NS = 2, 16
NW = NC * NS
ROWS_PER_W = T // NW
HALF_ROWS = 32

CHUNK = 4096
P_PAD = 1015808
NCHUNK = P_PAD // CHUNK
VEC_PER_CHUNK = CHUNK // 16
GROUPS = VEC_PER_CHUNK // 16

CAP = 48128


def _pack_body(n1_ref, n2_ref, d_ref, o_ref):
    o_ref[...] = (
        (n1_ref[...] << 16) | (n2_ref[...] << 5) | d_ref[...]
    )


def _bcast_body(p_ref, o_ref):
    o_ref[...] = jnp.broadcast_to(p_ref[...][None], o_ref.shape)


def _splat(x, u):
    return lax.gather(
        x, jnp.full((16, 1), u, jnp.int32),
        lax.GatherDimensionNumbers(
            offset_dims=(), collapsed_slice_dims=(0,), start_index_map=(0,)),
        (1,), mode=lax.GatherScatterMode.PROMISE_IN_BOUNDS)


def _sc_body(packed_hbm, emb_hbm, out_hbm, plist, buf0, buf1, scanbuf, embv,
             sems, osems):
    wid = lax.axis_index("s") * NC + lax.axis_index("c")
    row0 = wid * ROWS_PER_W

    pltpu.sync_copy(emb_hbm, embv)
    cvec = plsc.load_gather(embv, [jnp.full((16,), MAX_PATH - 1, jnp.int32)])
    iota = lax.iota(jnp.int32, 16)
    zero16 = jnp.zeros((16,), jnp.int32)

    neg1 = jnp.full((16,), -1, jnp.int32)

    def _pf(i, _):
        plist[pl.ds(i * 16, 16)] = neg1
        return 0

    lax.fori_loop(0, CAP // 16, _pf, 0)

    def _start(c):
        slot = c & 1
        return pltpu.async_copy(
            packed_hbm.at[pl.ds(c * CHUNK, CHUNK)], scanbuf.at[slot],
            sems.at[slot])

    _start(0)

    def _chunk(c, cntv):
        slot = c & 1

        @pl.when(c + 1 < NCHUNK)
        def _():
            _start(c + 1)

        pltpu.make_async_copy(
            packed_hbm.at[pl.ds(c * CHUNK, CHUNK)], scanbuf.at[slot],
            sems.at[slot]).wait()

        lo = row0 * 65536

        def _grp(g, cntv):
            vs = [scanbuf[slot, pl.ds((g * 16 + u) * 16, 16)]
                  for u in range(16)]
            ms = [(vs[u] - lo).astype(jnp.uint32)
                  < jnp.uint32(ROWS_PER_W * 65536) for u in range(16)]
            cnts = zero16
            for u in range(16):
                pc = plsc.all_reduce_population_count(ms[u])
                cnts = jnp.where(iota == u, pc, cnts)
            incl = plsc.cumsum(cnts)
            excl = incl - cnts
            for u in range(16):
                rank = plsc.cumsum(ms[u].astype(jnp.int32)) - 1
                base = jnp.minimum(cntv + _splat(excl, u), CAP - 16)
                plsc.store_scatter(plist, [base + rank], vs[u], mask=ms[u])
            return jnp.minimum(cntv + _splat(incl, 15), CAP - 16)

        return lax.fori_loop(0, GROUPS, _grp, cntv)

    cntv = lax.fori_loop(0, NCHUNK, _chunk, zero16)
    ngrp = (cntv[0] + 255) // 256

    bufs = [buf0, buf1]
    QROWS = ROWS_PER_W // 4

    def _drain(q):
        buf = bufs[q & 1]
        rowbase = row0 + q * QROWS
        for b in range(B):
            pltpu.make_async_copy(
                buf, out_hbm.at[b, pl.ds(rowbase, QROWS), :],
                osems.at[q & 1]).wait()

    for q in range(4):
        buf = bufs[q & 1]
        rowbase = row0 + q * QROWS
        if q >= 2:
            _drain(q - 2)

        def _init_row(i, _):
            def _init_v(j, _):
                for u in range(8):
                    buf[i, pl.ds(j * 128 + u * 16, 16)] = cvec
                return 0
            lax.fori_loop(0, T // 128, _init_v, 0)
            return 0

        lax.fori_loop(0, QROWS, _init_row, 0)

        def _scat(j, _):
            vs = [plist[pl.ds((j * 16 + u) * 16, 16)] for u in range(16)]
            rhs = [lax.shift_right_logical(vs[u], 16) - rowbase
                   for u in range(16)]
            ms = [rhs[u].astype(jnp.uint32) < jnp.uint32(QROWS)
                  for u in range(16)]
            n2s = [lax.shift_right_logical(vs[u], 5) & 0x7FF
                   for u in range(16)]
            vals = [plsc.load_gather(embv, [vs[u] & 0x1F])
                    for u in range(16)]
            for u in range(16):
                plsc.store_scatter(buf, [rhs[u], n2s[u]], vals[u],
                                   mask=ms[u])
            return 0

        lax.fori_loop(0, ngrp, _scat, 0)

        for b in range(B):
            pltpu.async_copy(
                buf, out_hbm.at[b, pl.ds(rowbase, QROWS), :],
                osems.at[q & 1])

    _drain(2)
    _drain(3)


@jax.jit
def kernel(x, path_distance_map, edge_index_map, distance_embedding):
    del x
    n1 = edge_index_map[:, 0]
    n2 = edge_index_map[:, 1]
    d = path_distance_map
    pad = P_PAD - P
    n1 = jnp.concatenate([n1, jnp.full((pad,), T, jnp.int32)]).reshape(-1, 128)
    n2 = jnp.concatenate([n2, jnp.zeros((pad,), jnp.int32)]).reshape(-1, 128)
    d = jnp.concatenate([d, jnp.zeros((pad,), jnp.int32)]).reshape(-1, 128)

    rows = P_PAD // 128
    blk = rows // 8
    packed = pl.pallas_call(
        _pack_body,
        out_shape=jax.ShapeDtypeStruct((rows, 128), jnp.int32),
        grid=(8,),
        in_specs=[pl.BlockSpec((blk, 128), lambda i: (i, 0))] * 3,
        out_specs=pl.BlockSpec((blk, 128), lambda i: (i, 0)),
    )(n1, n2, d).reshape(P_PAD)

    emb32 = jnp.concatenate(
        [distance_embedding[:, 0],
         jnp.zeros((32 - MAX_PATH,), jnp.float32)])

    sc = pl.kernel(
        _sc_body,
        out_type=jax.ShapeDtypeStruct((B, T, T), jnp.float32),
        mesh=plsc.VectorSubcoreMesh(core_axis_name="c", subcore_axis_name="s"),
        scratch_types=[
            pltpu.VMEM((CAP,), jnp.int32),
            pltpu.VMEM((ROWS_PER_W // 4, T), jnp.float32),
            pltpu.VMEM((ROWS_PER_W // 4, T), jnp.float32),
            pltpu.VMEM((2, CHUNK), jnp.int32),
            pltpu.VMEM((32,), jnp.float32),
            pltpu.SemaphoreType.DMA((2,)),
            pltpu.SemaphoreType.DMA((2,)),
        ],
        compiler_params=pltpu.CompilerParams(needs_layout_passes=False),
    )
    return sc(packed, emb32)
